# baseline (speedup 1.0000x reference)
import functools

import jax
import jax.numpy as jnp
from jax import lax
from jax.experimental import pallas as pl
from jax.experimental.pallas import tpu as pltpu
from jax.experimental.pallas import tpu_sc as plsc

NC = 2
NS = 16
L = 16
NW = NC * NS

B = 16384
D = 32
BPW = B // NW
CHUNK = 128
NCHUNK = BPW // CHUNK


def _dot_kernel(uidx_hbm, iidx_hbm, user_w_hbm, item_w_hbm, out_hbm,
                uidx_v, iidx_v, urows_v, vrows_v, out_v, usem, isem):
    wid = lax.axis_index("s") * NC + lax.axis_index("c")
    base = wid * BPW

    pltpu.sync_copy(uidx_hbm.at[wid], uidx_v)
    pltpu.sync_copy(iidx_hbm.at[wid], iidx_v)

    ucopies = []
    icopies = []
    for c in range(NCHUNK):
        ucopies.append(pltpu.async_copy(
            user_w_hbm.at[uidx_v.at[c]],
            urows_v.at[pl.ds(c * CHUNK, CHUNK)], usem))
        icopies.append(pltpu.async_copy(
            item_w_hbm.at[iidx_v.at[c]],
            vrows_v.at[pl.ds(c * CHUNK, CHUNK)], isem))
    for cp in ucopies:
        cp.wait()
    for cp in icopies:
        cp.wait()

    def body(g, _):
        rows = g * L + lax.iota(jnp.int32, L)
        acc = jnp.zeros((L,), jnp.float32)
        for d in range(D):
            cols = jnp.full((L,), d, jnp.int32)
            u = plsc.load_gather(urows_v, [rows, cols])
            v = plsc.load_gather(vrows_v, [rows, cols])
            acc = acc + u * v
        out_v[pl.ds(g * L, L)] = acc
        return 0

    lax.fori_loop(0, BPW // L, body, 0)

    pltpu.sync_copy(out_v, out_hbm.at[pl.ds(base, BPW)])


@jax.jit
def _run(user_idx, item_idx, user_w, item_w):
    mesh = plsc.VectorSubcoreMesh(core_axis_name="c", subcore_axis_name="s")
    k = functools.partial(
        pl.kernel,
        out_type=jax.ShapeDtypeStruct((B,), jnp.float32),
        mesh=mesh,
        compiler_params=pltpu.CompilerParams(
            needs_layout_passes=False, use_tc_tiling_on_sc=False),
        scratch_types=[
            pltpu.VMEM((NCHUNK, CHUNK), jnp.int32),
            pltpu.VMEM((NCHUNK, CHUNK), jnp.int32),
            pltpu.VMEM((BPW, D), jnp.float32),
            pltpu.VMEM((BPW, D), jnp.float32),
            pltpu.VMEM((BPW,), jnp.float32),
            pltpu.SemaphoreType.DMA,
            pltpu.SemaphoreType.DMA,
        ],
    )(_dot_kernel)
    uidx = user_idx.reshape(NW, NCHUNK, CHUNK)
    iidx = item_idx.reshape(NW, NCHUNK, CHUNK)
    return k(uidx, iidx, user_w, item_w)


def kernel(user_idx, item_idx, user_w, item_w):
    return _run(user_idx, item_idx, user_w, item_w)
